# baseline (speedup 1.0000x reference)
import jax
import jax.numpy as jnp
from jax.experimental import pallas as pl
from jax.experimental.pallas import tpu as pltpu

_B, _N, _D = 4, 4096, 3
_K = 8
_STRIP = 128
_NSTRIPS = _N // _STRIP
_YG = _N // 128


def _chamfer_body(x_ref, yr_ref, out_ref, wb_ref, macc_ref, xw_ref):
    b = pl.program_id(0)

    y0 = yr_ref[0, 0]
    y1 = yr_ref[0, 1]
    y2 = yr_ref[0, 2]
    ysq = y0 * y0 + y1 * y1 + y2 * y2
    wb_ref[0] = y0.astype(jnp.bfloat16)
    wb_ref[1] = y1.astype(jnp.bfloat16)
    wb_ref[2] = y2.astype(jnp.bfloat16)
    hi = ysq.astype(jnp.bfloat16)
    r1 = ysq - hi.astype(jnp.float32)
    mid = r1.astype(jnp.bfloat16)
    lo = (r1 - mid.astype(jnp.float32)).astype(jnp.bfloat16)
    wb_ref[3] = hi
    wb_ref[4] = mid
    wb_ref[5] = lo
    wb_ref[6] = jnp.zeros((_YG, 128), jnp.bfloat16)
    wb_ref[7] = jnp.zeros((_YG, 128), jnp.bfloat16)

    xw_ref[:, 0:_D] = (x_ref[0] * -2.0).astype(jnp.bfloat16)
    xw_ref[:, _D : 2 * _D] = jnp.ones((_N, _D), jnp.bfloat16)
    xw_ref[:, 2 * _D : _K] = jnp.zeros((_N, _K - 2 * _D), jnp.bfloat16)

    def strip_loop(s, carry):
        for half in range(2):
            base = (2 * s + half) * _STRIP
            xw = xw_ref[pl.ds(base, _STRIP), :]
            acc = None
            for m in range(_YG):
                w = wb_ref[:, m, :]
                t = jax.lax.dot_general(
                    xw, w, (((1,), (0,)), ((), ())),
                    preferred_element_type=jnp.float32,
                )
                acc = t if acc is None else jnp.minimum(acc, t)
            macc_ref[pl.ds(base, _STRIP), :] = acc
        return carry

    jax.lax.fori_loop(0, _NSTRIPS // 2, strip_loop, jnp.float32(0.0))

    m0 = jnp.min(macc_ref[...], axis=1, keepdims=True)
    xsf = x_ref[0]
    xsq = xsf[:, 0:1] * xsf[:, 0:1] + xsf[:, 1:2] * xsf[:, 1:2] \
        + xsf[:, 2:3] * xsf[:, 2:3]
    bsum = jnp.sum(jnp.maximum(m0 + xsq, 0.0))

    @pl.when(b == 0)
    def _init():
        out_ref[0, 0] = 0.0

    out_ref[0, 0] += bsum


def kernel(v, v_pred):
    yr = jnp.transpose(v, (0, 2, 1)).reshape(_B, _D, _YG, 128)
    out = pl.pallas_call(
        _chamfer_body,
        grid=(_B,),
        in_specs=[
            pl.BlockSpec((1, _N, _D), lambda b: (b, 0, 0)),
            pl.BlockSpec((1, _D, _YG, 128), lambda b: (b, 0, 0, 0)),
        ],
        out_specs=pl.BlockSpec(
            (1, 1), lambda b: (0, 0), memory_space=pltpu.SMEM
        ),
        out_shape=jax.ShapeDtypeStruct((1, 1), jnp.float32),
        scratch_shapes=[
            pltpu.VMEM((_K, _YG, 128), jnp.bfloat16),
            pltpu.VMEM((_N, 128), jnp.float32),
            pltpu.VMEM((_N, _K), jnp.bfloat16),
        ],
    )(v_pred, yr)
    return out[0, 0] * (1.0 / (_B * _N))


# speedup vs baseline: 1.1256x; 1.1256x over previous
import jax
import jax.numpy as jnp
from jax.experimental import pallas as pl
from jax.experimental.pallas import tpu as pltpu

_B, _N, _D = 4, 4096, 3
_K = 8
_NB = 512


def _chamfer_body(x_ref, y_ref, out_ref):
    b = pl.program_id(0)
    n = pl.program_id(1)

    y = y_ref[0]
    ysq = jnp.sum(y * y, axis=0, keepdims=True)
    hi = ysq.astype(jnp.bfloat16)
    r1 = ysq - hi.astype(jnp.float32)
    mid = r1.astype(jnp.bfloat16)
    lo = (r1 - mid.astype(jnp.float32)).astype(jnp.bfloat16)
    w = jnp.concatenate(
        [y.astype(jnp.bfloat16), hi, mid, lo,
         jnp.zeros((_K - 2 * _D, _N), jnp.bfloat16)], axis=0
    )

    x = x_ref[0]
    xw = jnp.concatenate(
        [(x * -2.0).astype(jnp.bfloat16),
         jnp.ones((_NB, _D), jnp.bfloat16),
         jnp.zeros((_NB, _K - 2 * _D), jnp.bfloat16)], axis=1
    )

    t = jax.lax.dot_general(
        xw, w, (((1,), (0,)), ((), ())),
        preferred_element_type=jnp.float32,
    )
    m = jnp.min(t, axis=1, keepdims=True)

    xsq = jnp.sum(x * x, axis=1, keepdims=True)
    bsum = jnp.sum(jnp.maximum(m + xsq, 0.0))

    @pl.when(jnp.logical_and(b == 0, n == 0))
    def _init():
        out_ref[0, 0] = 0.0

    out_ref[0, 0] += bsum


def kernel(v, v_pred):
    yT = jnp.transpose(v, (0, 2, 1))
    out = pl.pallas_call(
        _chamfer_body,
        grid=(_B, _N // _NB),
        in_specs=[
            pl.BlockSpec((1, _NB, _D), lambda b, n: (b, n, 0)),
            pl.BlockSpec((1, _D, _N), lambda b, n: (b, 0, 0)),
        ],
        out_specs=pl.BlockSpec(
            (1, 1), lambda b, n: (0, 0), memory_space=pltpu.SMEM
        ),
        out_shape=jax.ShapeDtypeStruct((1, 1), jnp.float32),
    )(v_pred, yT)
    return out[0, 0] * (1.0 / (_B * _N))


# speedup vs baseline: 1.2253x; 1.0886x over previous
import jax
import jax.numpy as jnp
from jax.experimental import pallas as pl
from jax.experimental.pallas import tpu as pltpu

_B, _N, _D = 4, 4096, 3
_K = 8
_NB = 1024


def _chamfer_body(x_ref, y_ref, out_ref):
    y = y_ref[0]
    ysq = jnp.sum(y * y, axis=0, keepdims=True)
    hi = ysq.astype(jnp.bfloat16)
    r1 = ysq - hi.astype(jnp.float32)
    mid = r1.astype(jnp.bfloat16)
    lo = (r1 - mid.astype(jnp.float32)).astype(jnp.bfloat16)
    w = jnp.concatenate(
        [y.astype(jnp.bfloat16), hi, mid, lo,
         jnp.zeros((_K - 2 * _D, _N), jnp.bfloat16)], axis=0
    )

    x = x_ref[0]
    xw = jnp.concatenate(
        [(x * -2.0).astype(jnp.bfloat16),
         jnp.ones((_NB, _D), jnp.bfloat16),
         jnp.zeros((_NB, _K - 2 * _D), jnp.bfloat16)], axis=1
    )

    t = jax.lax.dot_general(
        xw, w, (((1,), (0,)), ((), ())),
        preferred_element_type=jnp.float32,
    )
    m = jnp.min(t, axis=1, keepdims=True)

    xsq = jnp.sum(x * x, axis=1, keepdims=True)
    bsum = jnp.sum(jnp.maximum(m + xsq, 0.0))
    out_ref[...] = jnp.full((8, 128), bsum, jnp.float32)


def kernel(v, v_pred):
    yT = jnp.transpose(v, (0, 2, 1))
    out = pl.pallas_call(
        _chamfer_body,
        grid=(_B, _N // _NB),
        in_specs=[
            pl.BlockSpec((1, _NB, _D), lambda b, n: (b, n, 0)),
            pl.BlockSpec((1, _D, _N), lambda b, n: (b, 0, 0)),
        ],
        out_specs=pl.BlockSpec((8, 128), lambda b, n: (b, n)),
        out_shape=jax.ShapeDtypeStruct(
            (_B * 8, (_N // _NB) * 128), jnp.float32
        ),
        compiler_params=pltpu.CompilerParams(
            dimension_semantics=("parallel", "parallel"),
        ),
    )(v_pred, yT)
    return jnp.sum(out) * (1.0 / (_B * _N * 8 * 128))
